# Initial kernel scaffold; baseline (speedup 1.0000x reference)
import jax
import jax.numpy as jnp
from jax.experimental import pallas as pl


def kernel(x, edge_index, edge_weight, W, b, prelu_alpha):
    raise NotImplementedError("write your pallas kernel here")


# speedup vs baseline: 16.7473x; 16.7473x over previous
import dataclasses
import functools

import jax
import jax.numpy as jnp
from jax import lax
from jax.experimental import pallas as pl
from jax.experimental.pallas import tpu as pltpu
from jax.experimental.pallas import tpu_sc as plsc

N, E, F_IN, H = 10000, 320000, 128, 128
NC, NS = 2, 16
NW = NC * NS
CHUNK = 128
EP_TILE = 10112
KCHUNKS = EP_TILE // CHUNK
E_PAD = NW * EP_TILE
N_PAD = 10240
ROWS_TILE = N_PAD // NS
ZROWS = 8

def _sc_degree_body(col_hbm, w_hbm, deg_hbm, acc, colbuf, wbuf, zbuf):
    cid = lax.axis_index("c")
    sid = lax.axis_index("s")
    wid = cid * NS + sid

    @pl.loop(0, (N_PAD // NS) // 16)
    def _(i):
        zbuf[pl.ds(i * 16, 16)] = jnp.zeros((16,), jnp.float32)

    pltpu.sync_copy(zbuf, acc.at[pl.ds(sid * (N_PAD // NS), N_PAD // NS)])
    plsc.subcore_barrier()

    pltpu.sync_copy(col_hbm.at[wid], colbuf)
    pltpu.sync_copy(w_hbm.at[wid], wbuf)

    @pl.loop(0, KCHUNKS)
    def _(k):
        pltpu.sync_copy(wbuf.at[k], acc.at[colbuf.at[k]], add=True)

    plsc.subcore_barrier()
    pltpu.sync_copy(
        acc.at[pl.ds(sid * (N_PAD // NS), N_PAD // NS)],
        deg_hbm.at[cid, pl.ds(sid * (N_PAD // NS), N_PAD // NS)],
    )


def _sc_aggregate_body(row_hbm, col_hbm, w_hbm, h2_hbm, out_hbm,
                       acc, rowbuf, colbuf, wbuf, rows, zbuf):
    cid = lax.axis_index("c")
    sid = lax.axis_index("s")
    wid = cid * NS + sid

    @pl.loop(0, ZROWS)
    def _(r):
        for j in range(H // 16):
            zbuf[r, pl.ds(j * 16, 16)] = jnp.zeros((16,), jnp.float32)

    @pl.loop(0, ROWS_TILE // ZROWS)
    def _(zi):
        pltpu.sync_copy(zbuf, acc.at[pl.ds(sid * ROWS_TILE + zi * ZROWS, ZROWS)])

    plsc.subcore_barrier()

    pltpu.sync_copy(row_hbm.at[wid], rowbuf)
    pltpu.sync_copy(col_hbm.at[wid], colbuf)
    pltpu.sync_copy(w_hbm.at[wid], wbuf)

    @pl.loop(0, KCHUNKS)
    def _(k):
        pltpu.sync_copy(h2_hbm.at[rowbuf.at[k]], rows)

        @pl.loop(0, CHUNK)
        def _(e):
            wv = plsc.load_gather(
                wbuf, [jnp.full((16,), k, jnp.int32), jnp.full((16,), e, jnp.int32)]
            )
            for j in range(H // 16):
                rows[e, pl.ds(j * 16, 16)] = rows[e, pl.ds(j * 16, 16)] * wv

        pltpu.sync_copy(rows, acc.at[colbuf.at[k]], add=True)

    plsc.subcore_barrier()
    pltpu.sync_copy(
        acc.at[pl.ds(sid * ROWS_TILE, ROWS_TILE)],
        out_hbm.at[cid, pl.ds(sid * ROWS_TILE, ROWS_TILE)],
    )


@functools.lru_cache(maxsize=1)
def _build_sc_kernels():
    mesh = plsc.VectorSubcoreMesh(core_axis_name="c", subcore_axis_name="s")
    cp = pltpu.CompilerParams()
    if "needs_layout_passes" in pltpu.CompilerParams.__dataclass_fields__:
        cp = dataclasses.replace(cp, needs_layout_passes=False)
    sc_degree = functools.partial(
        pl.kernel,
        out_type=jax.ShapeDtypeStruct((NC, N_PAD), jnp.float32),
        mesh=mesh,
        scratch_types=[
            pltpu.VMEM_SHARED((N_PAD,), jnp.float32),
            pltpu.VMEM((KCHUNKS, CHUNK), jnp.int32),
            pltpu.VMEM((KCHUNKS, CHUNK), jnp.float32),
            pltpu.VMEM((N_PAD // NS,), jnp.float32),
        ],
    )(_sc_degree_body)
    sc_aggregate = functools.partial(
        pl.kernel,
        out_type=jax.ShapeDtypeStruct((NC, N_PAD, H), jnp.float32),
        mesh=mesh,
        scratch_types=[
            pltpu.VMEM_SHARED((N_PAD, H), jnp.float32),
            pltpu.VMEM((KCHUNKS, CHUNK), jnp.int32),
            pltpu.VMEM((KCHUNKS, CHUNK), jnp.int32),
            pltpu.VMEM((KCHUNKS, CHUNK), jnp.float32),
            pltpu.VMEM((CHUNK, H), jnp.float32),
            pltpu.VMEM((ZROWS, H), jnp.float32),
        ],
        compiler_params=cp,
    )(_sc_aggregate_body)
    return sc_degree, sc_aggregate


def _tc_matmul_body(x_ref, w_ref, o_ref):
    o_ref[...] = jnp.dot(x_ref[...], w_ref[...],
                         preferred_element_type=jnp.float32)


def _tc_dis_body(d0_ref, d1_ref, o_ref):
    deg = d0_ref[...] + d1_ref[...] + 1.0
    o_ref[...] = jnp.where(deg > 0, lax.rsqrt(jnp.where(deg > 0, deg, 1.0)), 0.0)


def _tc_scale_body(dis_ref, h_ref, o_ref):
    o_ref[...] = dis_ref[...] * h_ref[...]


def _tc_final_body(acc_ref, h2_ref, dis_ref, b_ref, a_ref, o_ref):
    s = dis_ref[...] * (acc_ref[0, :N] + acc_ref[1, :N] + h2_ref[...]) + b_ref[...]
    o_ref[...] = jnp.where(s >= 0, s, a_ref[...] * s)


def kernel(x, edge_index, edge_weight, W, b, prelu_alpha):
    row, col = edge_index[0], edge_index[1]
    pad = E_PAD - E
    row_p = jnp.concatenate([row, jnp.zeros((pad,), jnp.int32)]).reshape(NW, KCHUNKS, CHUNK)
    col_p = jnp.concatenate([col, jnp.zeros((pad,), jnp.int32)]).reshape(NW, KCHUNKS, CHUNK)
    w_p = jnp.concatenate([edge_weight, jnp.zeros((pad,), jnp.float32)]).reshape(NW, KCHUNKS, CHUNK)

    sc_degree, sc_aggregate = _build_sc_kernels()
    deg_parts = sc_degree(col_p, w_p)
    h = pl.pallas_call(
        _tc_matmul_body,
        out_shape=jax.ShapeDtypeStruct((N, H), jnp.float32),
    )(x, W)

    dis2d = pl.pallas_call(
        _tc_dis_body,
        out_shape=jax.ShapeDtypeStruct((N_PAD // 128, 128), jnp.float32),
    )(deg_parts[0].reshape(N_PAD // 128, 128),
      deg_parts[1].reshape(N_PAD // 128, 128))
    dis_col = dis2d.reshape(N_PAD)[:N][:, None]

    h2 = pl.pallas_call(
        _tc_scale_body,
        out_shape=jax.ShapeDtypeStruct((N, H), jnp.float32),
    )(dis_col, h)

    acc_parts = sc_aggregate(row_p, col_p, w_p, h2)

    out = pl.pallas_call(
        _tc_final_body,
        out_shape=jax.ShapeDtypeStruct((N, H), jnp.float32),
    )(acc_parts, h2, dis_col, b.reshape(1, H),
      prelu_alpha.reshape(1, 1))
    return out


# speedup vs baseline: 17.2583x; 1.0305x over previous
import dataclasses
import functools

import jax
import jax.numpy as jnp
from jax import lax
from jax.experimental import pallas as pl
from jax.experimental.pallas import tpu as pltpu
from jax.experimental.pallas import tpu_sc as plsc

N, E, F_IN, H = 10000, 320000, 128, 128
NC, NS = 2, 16
NW = NC * NS
CHUNK = 128
EP_TILE = 10112
KCHUNKS = EP_TILE // CHUNK
E_PAD = NW * EP_TILE
N_PAD = 10240
ROWS_TILE = N_PAD // NS
ZROWS = 8

def _sc_degree_body(col_hbm, w_hbm, deg_hbm, acc, colbuf, wbuf, zbuf):
    cid = lax.axis_index("c")
    sid = lax.axis_index("s")
    wid = cid * NS + sid

    @pl.loop(0, (N_PAD // NS) // 16)
    def _(i):
        zbuf[pl.ds(i * 16, 16)] = jnp.zeros((16,), jnp.float32)

    pltpu.sync_copy(zbuf, acc.at[pl.ds(sid * (N_PAD // NS), N_PAD // NS)])
    plsc.subcore_barrier()

    pltpu.sync_copy(col_hbm.at[wid], colbuf)
    pltpu.sync_copy(w_hbm.at[wid], wbuf)

    @pl.loop(0, KCHUNKS)
    def _(k):
        pltpu.sync_copy(wbuf.at[k], acc.at[colbuf.at[k]], add=True)

    plsc.subcore_barrier()
    pltpu.sync_copy(
        acc.at[pl.ds(sid * (N_PAD // NS), N_PAD // NS)],
        deg_hbm.at[cid, pl.ds(sid * (N_PAD // NS), N_PAD // NS)],
    )


def _sc_aggregate_body(row_hbm, col_hbm, w_hbm, h2_hbm, out_hbm,
                       acc, rowbuf, colbuf, wbuf, rows):
    cid = lax.axis_index("c")
    sid = lax.axis_index("s")
    wid = cid * NS + sid

    @pl.loop(0, CHUNK, unroll=4)
    def _(r):
        for j in range(H // 16):
            rows[r, pl.ds(j * 16, 16)] = jnp.zeros((16,), jnp.float32)

    @pl.loop(0, ROWS_TILE // CHUNK)
    def _(zi):
        pltpu.sync_copy(rows, acc.at[pl.ds(sid * ROWS_TILE + zi * CHUNK, CHUNK)])

    plsc.subcore_barrier()

    pltpu.sync_copy(row_hbm.at[wid], rowbuf)
    pltpu.sync_copy(col_hbm.at[wid], colbuf)
    pltpu.sync_copy(w_hbm.at[wid], wbuf)

    @pl.loop(0, KCHUNKS)
    def _(k):
        pltpu.sync_copy(h2_hbm.at[rowbuf.at[k]], rows)

        @pl.loop(0, CHUNK, unroll=4)
        def _(e):
            wv = plsc.load_gather(
                wbuf, [jnp.full((16,), k, jnp.int32), jnp.full((16,), e, jnp.int32)]
            )
            for j in range(H // 16):
                rows[e, pl.ds(j * 16, 16)] = rows[e, pl.ds(j * 16, 16)] * wv

        pltpu.sync_copy(rows, acc.at[colbuf.at[k]], add=True)

    plsc.subcore_barrier()
    pltpu.sync_copy(
        acc.at[pl.ds(sid * ROWS_TILE, ROWS_TILE)],
        out_hbm.at[cid, pl.ds(sid * ROWS_TILE, ROWS_TILE)],
    )


@functools.lru_cache(maxsize=1)
def _build_sc_kernels():
    mesh = plsc.VectorSubcoreMesh(core_axis_name="c", subcore_axis_name="s")
    cp = pltpu.CompilerParams()
    if "needs_layout_passes" in pltpu.CompilerParams.__dataclass_fields__:
        cp = dataclasses.replace(cp, needs_layout_passes=False)
    sc_degree = functools.partial(
        pl.kernel,
        out_type=jax.ShapeDtypeStruct((NC, N_PAD), jnp.float32),
        mesh=mesh,
        scratch_types=[
            pltpu.VMEM_SHARED((N_PAD,), jnp.float32),
            pltpu.VMEM((KCHUNKS, CHUNK), jnp.int32),
            pltpu.VMEM((KCHUNKS, CHUNK), jnp.float32),
            pltpu.VMEM((N_PAD // NS,), jnp.float32),
        ],
    )(_sc_degree_body)
    sc_aggregate = functools.partial(
        pl.kernel,
        out_type=jax.ShapeDtypeStruct((NC, N_PAD, H), jnp.float32),
        mesh=mesh,
        scratch_types=[
            pltpu.VMEM_SHARED((N_PAD, H), jnp.float32),
            pltpu.VMEM((KCHUNKS, CHUNK), jnp.int32),
            pltpu.VMEM((KCHUNKS, CHUNK), jnp.int32),
            pltpu.VMEM((KCHUNKS, CHUNK), jnp.float32),
            pltpu.VMEM((CHUNK, H), jnp.float32),
        ],
        compiler_params=cp,
    )(_sc_aggregate_body)
    return sc_degree, sc_aggregate


def _tc_matmul_body(x_ref, w_ref, o_ref):
    o_ref[...] = jnp.dot(x_ref[...], w_ref[...],
                         preferred_element_type=jnp.float32)


def _tc_dis_body(d0_ref, d1_ref, o_ref):
    deg = d0_ref[...] + d1_ref[...] + 1.0
    o_ref[...] = jnp.where(deg > 0, lax.rsqrt(jnp.where(deg > 0, deg, 1.0)), 0.0)


def _tc_scale_body(dis_ref, h_ref, o_ref):
    o_ref[...] = dis_ref[...] * h_ref[...]


def _tc_final_body(acc_ref, h2_ref, dis_ref, b_ref, a_ref, o_ref):
    s = dis_ref[...] * (acc_ref[0, :N] + acc_ref[1, :N] + h2_ref[...]) + b_ref[...]
    o_ref[...] = jnp.where(s >= 0, s, a_ref[...] * s)


def kernel(x, edge_index, edge_weight, W, b, prelu_alpha):
    row, col = edge_index[0], edge_index[1]
    pad = E_PAD - E
    row_p = jnp.concatenate([row, jnp.zeros((pad,), jnp.int32)]).reshape(NW, KCHUNKS, CHUNK)
    col_p = jnp.concatenate([col, jnp.zeros((pad,), jnp.int32)]).reshape(NW, KCHUNKS, CHUNK)
    w_p = jnp.concatenate([edge_weight, jnp.zeros((pad,), jnp.float32)]).reshape(NW, KCHUNKS, CHUNK)

    sc_degree, sc_aggregate = _build_sc_kernels()
    deg_parts = sc_degree(col_p, w_p)
    h = pl.pallas_call(
        _tc_matmul_body,
        out_shape=jax.ShapeDtypeStruct((N, H), jnp.float32),
    )(x, W)

    dis2d = pl.pallas_call(
        _tc_dis_body,
        out_shape=jax.ShapeDtypeStruct((N_PAD // 128, 128), jnp.float32),
    )(deg_parts[0].reshape(N_PAD // 128, 128),
      deg_parts[1].reshape(N_PAD // 128, 128))
    dis_col = dis2d.reshape(N_PAD)[:N][:, None]

    h2 = pl.pallas_call(
        _tc_scale_body,
        out_shape=jax.ShapeDtypeStruct((N, H), jnp.float32),
    )(dis_col, h)

    acc_parts = sc_aggregate(row_p, col_p, w_p, h2)

    out = pl.pallas_call(
        _tc_final_body,
        out_shape=jax.ShapeDtypeStruct((N, H), jnp.float32),
    )(acc_parts, h2, dis_col, b.reshape(1, H),
      prelu_alpha.reshape(1, 1))
    return out
